# Initial kernel scaffold; baseline (speedup 1.0000x reference)
import jax
import jax.numpy as jnp
from jax.experimental import pallas as pl


def kernel(x, size, attention_mask, query, ln_q_w, ln_q_b, ln_k_w, ln_k_b, Wq, Wk, Wv, bq, bk, bv, Wo, bo):
    raise NotImplementedError("write your pallas kernel here")


# speedup vs baseline: 2.2019x; 2.2019x over previous
import jax
import jax.numpy as jnp
from jax.experimental import pallas as pl
from jax.experimental.pallas import tpu as pltpu

D_M = 768
C_K = 1024
N_H = 8
H_D = 96
H_P = 128
D_P = N_H * H_P
N_Q = 256
EPS_LN = 1e-5
SCALE = 1.0 / (H_D ** 0.5)


def _q_proj_kernel(q_ref, lnw_ref, lnb_ref, wq_ref, bq_ref, out_ref):
    q = q_ref[...]
    mu = jnp.mean(q, axis=-1, keepdims=True)
    var = jnp.mean((q - mu) ** 2, axis=-1, keepdims=True)
    qn = (q - mu) * jax.lax.rsqrt(var + EPS_LN) * lnw_ref[...] + lnb_ref[...]
    qh = jnp.dot(qn, wq_ref[...], preferred_element_type=jnp.float32)
    out_ref[...] = (qh + bq_ref[...]) * SCALE


def _pool_kernel(x_ref, size_ref, mask_ref, lnw_ref, lnb_ref, qh_ref,
                 wk_ref, bk_ref, wv_ref, bv_ref, wo_ref, bo_ref,
                 out_ref, kh_s, vh_s, oacc_s):
    xb = x_ref[0]
    mu = jnp.mean(xb, axis=-1, keepdims=True)
    var = jnp.mean((xb - mu) ** 2, axis=-1, keepdims=True)
    xk = (xb - mu) * jax.lax.rsqrt(var + EPS_LN) * lnw_ref[...] + lnb_ref[...]
    kh_s[...] = jnp.dot(xk, wk_ref[...], preferred_element_type=jnp.float32) + bk_ref[...]
    vh_s[...] = jnp.dot(xk, wv_ref[...], preferred_element_type=jnp.float32) + bv_ref[...]
    sz = size_ref[0]
    bias = jnp.log(jnp.where(sz < 0.5, 1.0, sz)) + mask_ref[0]
    for h in range(N_H):
        lo = h * H_P
        s = jax.lax.dot_general(qh_ref[:, lo:lo + H_P], kh_s[:, lo:lo + H_P],
                                (((1,), (1,)), ((), ())),
                                preferred_element_type=jnp.float32)
        s = s + bias
        m = jnp.max(s, axis=-1, keepdims=True)
        p = jnp.exp(s - m)
        denom = jnp.sum(p, axis=-1, keepdims=True)
        o_h = jnp.dot(p, vh_s[:, lo:lo + H_P], preferred_element_type=jnp.float32)
        oacc_s[:, lo:lo + H_P] = o_h / denom
    out_ref[0] = jnp.dot(oacc_s[...], wo_ref[...],
                         preferred_element_type=jnp.float32) + bo_ref[...]


def _pad_heads_cols(w):
    n = w.shape[0]
    return jnp.pad(w.reshape(n, N_H, H_D), ((0, 0), (0, 0), (0, H_P - H_D))).reshape(n, D_P)


@jax.jit
def kernel(x, size, attention_mask, query, ln_q_w, ln_q_b, ln_k_w, ln_k_b,
           Wq, Wk, Wv, bq, bk, bv, Wo, bo):
    B, L, _ = x.shape
    f32 = jnp.float32

    wq_p = _pad_heads_cols(Wq.T)
    wk_p = _pad_heads_cols(Wk.T)
    wv_p = _pad_heads_cols(Wv.T)
    bq_p = jnp.pad(bq.reshape(N_H, H_D), ((0, 0), (0, H_P - H_D))).reshape(1, D_P)
    bk_p = jnp.pad(bk.reshape(N_H, H_D), ((0, 0), (0, H_P - H_D))).reshape(1, D_P)
    bv_p = jnp.pad(bv.reshape(N_H, H_D), ((0, 0), (0, H_P - H_D))).reshape(1, D_P)
    wo_p = jnp.pad(Wo.T.reshape(N_H, H_D, D_M), ((0, 0), (0, H_P - H_D), (0, 0))).reshape(D_P, D_M)

    qh_pad = pl.pallas_call(
        _q_proj_kernel,
        out_shape=jax.ShapeDtypeStruct((N_Q, D_P), f32),
        name="q_proj",
    )(query, ln_q_w.reshape(1, D_M), ln_q_b.reshape(1, D_M), wq_p, bq_p)

    full = lambda shape: pl.BlockSpec(shape, lambda b: (0,) * len(shape))
    out = pl.pallas_call(
        _pool_kernel,
        grid=(B,),
        in_specs=[
            pl.BlockSpec((1, L, C_K), lambda b: (b, 0, 0)),
            pl.BlockSpec((1, 1, L), lambda b: (b, 0, 0)),
            pl.BlockSpec((1, 1, L), lambda b: (b, 0, 0)),
            full((1, C_K)),
            full((1, C_K)),
            full((N_Q, D_P)),
            full((C_K, D_P)),
            full((1, D_P)),
            full((C_K, D_P)),
            full((1, D_P)),
            full((D_P, D_M)),
            full((1, D_M)),
        ],
        out_specs=pl.BlockSpec((1, N_Q, D_M), lambda b: (b, 0, 0)),
        out_shape=jax.ShapeDtypeStruct((B, N_Q, D_M), f32),
        scratch_shapes=[
            pltpu.VMEM((L, D_P), f32),
            pltpu.VMEM((L, D_P), f32),
            pltpu.VMEM((N_Q, D_P), f32),
        ],
        compiler_params=pltpu.CompilerParams(
            dimension_semantics=("parallel",),
            vmem_limit_bytes=56 * 1024 * 1024,
        ),
        name="attn_pool",
    )(x, size[:, :, 0][:, None, :], attention_mask,
      ln_k_w.reshape(1, C_K), ln_k_b.reshape(1, C_K), qh_pad,
      wk_p, bk_p, wv_p, bv_p, wo_p, bo.reshape(1, D_M))
    return out
